# speedup vs baseline: 2.3761x; 2.3761x over previous
import functools

import jax
import jax.numpy as jnp
from jax.experimental import pallas as pl
from jax.experimental.pallas import tpu as pltpu

N_NODES = 10000
N_EDGES = 320000


def _mm_body(a_ref, w_ref, b_ref, o_ref):
    o_ref[...] = (
        jnp.dot(a_ref[...], w_ref[...], preferred_element_type=jnp.float32)
        + b_ref[...]
    )


def _mm(a, w, b):
    m, k = a.shape
    n = w.shape[1]
    return pl.pallas_call(
        _mm_body,
        out_shape=jax.ShapeDtypeStruct((m, n), jnp.float32),
    )(a, w, b[None, :])


def kernel(x, edge_index, edge_attr, W1, b1, W2, b2, W3, b3):
    N = x.shape[0]
    src = edge_index[0].astype(jnp.int32)
    dst = edge_index[1].astype(jnp.int32)
    ew = edge_attr

    deg = jax.ops.segment_sum(ew, dst, num_segments=N) + 1.0
    dis = jax.lax.rsqrt(deg)

    def layer(h_in, W, b, relu):
        hp = dis[:, None] * _mm(h_in, W, jnp.zeros((W.shape[1],), jnp.float32))
        msg = hp[src] * ew[:, None]
        acc = jax.ops.segment_sum(msg, dst, num_segments=N)
        z = dis[:, None] * (acc + hp) + b
        return jax.nn.relu(z) if relu else z

    h = layer(x, W1, b1, True)
    h = layer(h, W2, b2, True)
    return layer(h, W3, b3, False)


# speedup vs baseline: 8.7137x; 3.6673x over previous
import functools

import jax
import jax.numpy as jnp
from jax import lax
from jax.experimental import pallas as pl
from jax.experimental.pallas import tpu as pltpu
from jax.experimental.pallas import tpu_sc as plsc

N = 10000
N_PAD = 10240
E = 320000
NC = 2
NS = 16
K = 128
CPT = 79
E_PAD = NC * NS * CPT * K
RPT = N_PAD // NS
RCH = 128
RN = RPT // RCH

_MESH = plsc.VectorSubcoreMesh(core_axis_name="c", subcore_axis_name="s")
_SC_PARAMS = pltpu.CompilerParams(use_tc_tiling_on_sc=False)


def _zero16():
    return jnp.zeros((16,), jnp.float32)


def _deg_body(dst_hbm, ew_hbm, out0, out1, dst_v, ew_v, acc):
    c = lax.axis_index("c")
    s = lax.axis_index("s")
    wid = s * NC + c
    r0 = s * RPT
    for j in range(K // 16):
        ew_v[pl.ds(j * 16, 16)] = _zero16()
    for j in range(RN):
        pltpu.sync_copy(ew_v, acc.at[pl.ds(r0 + j * RCH, RCH)])
    plsc.subcore_barrier()

    def chunk(i, carry):
        base = (wid * CPT + i) * K
        pltpu.sync_copy(dst_hbm.at[pl.ds(base, K)], dst_v)
        pltpu.sync_copy(ew_hbm.at[pl.ds(base, K)], ew_v)
        pltpu.sync_copy(ew_v, acc.at[dst_v], add=True)
        return carry

    lax.fori_loop(0, CPT, chunk, 0)
    plsc.subcore_barrier()

    @pl.when(c == 0)
    def _():
        for j in range(RN):
            sl = pl.ds(r0 + j * RCH, RCH)
            pltpu.sync_copy(acc.at[sl], out0.at[sl])

    @pl.when(c == 1)
    def _():
        for j in range(RN):
            sl = pl.ds(r0 + j * RCH, RCH)
            pltpu.sync_copy(acc.at[sl], out1.at[sl])


_deg = pl.kernel(
    _deg_body,
    out_type=[jax.ShapeDtypeStruct((N_PAD,), jnp.float32)] * 2,
    mesh=_MESH,
    scratch_types=[
        pltpu.VMEM((K,), jnp.int32),
        pltpu.VMEM((K,), jnp.float32),
        pltpu.VMEM_SHARED((N_PAD,), jnp.float32),
    ],
    compiler_params=_SC_PARAMS,
)


def _make_scatter(F):
    nz = F // 16

    def body(hp_hbm, src_hbm, dst_hbm, ew_hbm, out0, out1,
             src_v, dst_v, ew_v, rows_v, acc, sem):
        c = lax.axis_index("c")
        s = lax.axis_index("s")
        wid = s * NC + c
        r0 = s * RPT

        def zrow(e, carry):
            for j in range(nz):
                rows_v[e, pl.ds(j * 16, 16)] = _zero16()
            return carry

        lax.fori_loop(0, K, zrow, 0)
        for j in range(RN):
            pltpu.sync_copy(rows_v, acc.at[pl.ds(r0 + j * RCH, RCH)])
        plsc.subcore_barrier()

        def chunk(i, carry):
            base = (wid * CPT + i) * K
            pltpu.sync_copy(src_hbm.at[pl.ds(base, K)], src_v)
            pltpu.sync_copy(dst_hbm.at[pl.ds(base, K)], dst_v)
            pltpu.sync_copy(ew_hbm.at[pl.ds(base, K)], ew_v)
            pltpu.async_copy(hp_hbm.at[src_v], rows_v, sem).wait()

            def escale(g, cc):
                w16 = ew_v[pl.ds(g * 16, 16)]
                for l in range(16):
                    w = w16[l]
                    e = g * 16 + l
                    for j in range(nz):
                        sl = pl.ds(j * 16, 16)
                        rows_v[e, sl] = rows_v[e, sl] * w
                return cc

            lax.fori_loop(0, K // 16, escale, 0)
            pltpu.sync_copy(rows_v, acc.at[dst_v], add=True)
            return carry

        lax.fori_loop(0, CPT, chunk, 0)
        plsc.subcore_barrier()

        @pl.when(c == 0)
        def _():
            for j in range(RN):
                sl = pl.ds(r0 + j * RCH, RCH)
                pltpu.sync_copy(acc.at[sl], out0.at[sl])

        @pl.when(c == 1)
        def _():
            for j in range(RN):
                sl = pl.ds(r0 + j * RCH, RCH)
                pltpu.sync_copy(acc.at[sl], out1.at[sl])

    return pl.kernel(
        body,
        out_type=[jax.ShapeDtypeStruct((N_PAD, F), jnp.float32)] * 2,
        mesh=_MESH,
        scratch_types=[
            pltpu.VMEM((K,), jnp.int32),
            pltpu.VMEM((K,), jnp.int32),
            pltpu.VMEM((K,), jnp.float32),
            pltpu.VMEM((K, F), jnp.float32),
            pltpu.VMEM_SHARED((N_PAD, F), jnp.float32),
            pltpu.SemaphoreType.DMA,
        ],
        compiler_params=_SC_PARAMS,
    )


_scatter128 = _make_scatter(128)
_scatter64 = _make_scatter(64)
_scatter32 = _make_scatter(32)


def _dis_body(d0_ref, d1_ref, o_ref):
    o_ref[...] = lax.rsqrt(d0_ref[...] + d1_ref[...] + 1.0)


def _dis(d0, d1):
    return pl.pallas_call(
        _dis_body,
        out_shape=jax.ShapeDtypeStruct((N_PAD, 1), jnp.float32),
    )(d0.reshape(N_PAD, 1), d1.reshape(N_PAD, 1))


def _k1_body(x_ref, w_ref, dis_ref, o_ref):
    dis = dis_ref[pl.ds(0, N), :]
    o_ref[...] = (
        jnp.dot(x_ref[...], w_ref[...], preferred_element_type=jnp.float32)
        * dis
    )


def _k1(x, W, dis):
    return pl.pallas_call(
        _k1_body,
        out_shape=jax.ShapeDtypeStruct((N, W.shape[1]), jnp.float32),
    )(x, W, dis)


def _fuse_body(a0_ref, a1_ref, hp_ref, dis_ref, b_ref, w_ref, o_ref):
    dis = dis_ref[pl.ds(0, N), :]
    acc = a0_ref[pl.ds(0, N), :] + a1_ref[pl.ds(0, N), :]
    z = dis * (acc + hp_ref[...]) + b_ref[...]
    a = jnp.maximum(z, 0.0)
    o_ref[...] = (
        jnp.dot(a, w_ref[...], preferred_element_type=jnp.float32) * dis
    )


def _fuse(a0, a1, hp, dis, b, W):
    return pl.pallas_call(
        _fuse_body,
        out_shape=jax.ShapeDtypeStruct((N, W.shape[1]), jnp.float32),
    )(a0, a1, hp, dis, b[None, :], W)


def _final_body(a0_ref, a1_ref, hp_ref, dis_ref, b_ref, o_ref):
    dis = dis_ref[pl.ds(0, N), :]
    acc = a0_ref[pl.ds(0, N), :] + a1_ref[pl.ds(0, N), :]
    o_ref[...] = dis * (acc + hp_ref[...]) + b_ref[...]


def _final(a0, a1, hp, dis, b):
    return pl.pallas_call(
        _final_body,
        out_shape=jax.ShapeDtypeStruct((N, b.shape[0]), jnp.float32),
    )(a0, a1, hp, dis, b[None, :])


def kernel(x, edge_index, edge_attr, W1, b1, W2, b2, W3, b3):
    pad = E_PAD - E
    src = jnp.concatenate([edge_index[0].astype(jnp.int32),
                           jnp.zeros((pad,), jnp.int32)])
    dst = jnp.concatenate([edge_index[1].astype(jnp.int32),
                           jnp.zeros((pad,), jnp.int32)])
    ew = jnp.concatenate([edge_attr, jnp.zeros((pad,), jnp.float32)])

    d0, d1 = _deg(dst, ew)
    dis = _dis(d0, d1)

    hp1 = _k1(x, W1, dis)
    a0, a1 = _scatter128(hp1, src, dst, ew)
    hp2 = _fuse(a0, a1, hp1, dis, b1, W2)
    a0, a1 = _scatter64(hp2, src, dst, ew)
    hp3 = _fuse(a0, a1, hp2, dis, b2, W3)
    a0, a1 = _scatter32(hp3, src, dst, ew)
    return _final(a0, a1, hp3, dis, b3)


# speedup vs baseline: 8.8595x; 1.0167x over previous
import functools

import jax
import jax.numpy as jnp
from jax import lax
from jax.experimental import pallas as pl
from jax.experimental.pallas import tpu as pltpu
from jax.experimental.pallas import tpu_sc as plsc

N = 10000
N_PAD = 10240
E = 320000
NC = 2
NS = 16
K = 128
CPT = 80
E_PAD = NC * NS * CPT * K
NCHUNKS = NC * NS * CPT
RPT = N_PAD // NS
RCH = 128
RN = RPT // RCH

_MESH = plsc.VectorSubcoreMesh(core_axis_name="c", subcore_axis_name="s")
_SC_PARAMS = pltpu.CompilerParams(use_tc_tiling_on_sc=False)


def _zero16():
    return jnp.zeros((16,), jnp.float32)


def _deg_body(dst_hbm, ew_hbm, out0, out1, dst_a, ew_a, zb, acc):
    c = lax.axis_index("c")
    s = lax.axis_index("s")
    wid = s * NC + c
    r0 = s * RPT
    pltpu.sync_copy(dst_hbm.at[pl.ds(wid * CPT, CPT)], dst_a)
    pltpu.sync_copy(ew_hbm.at[pl.ds(wid * CPT, CPT)], ew_a)
    for j in range(K // 16):
        zb[pl.ds(j * 16, 16)] = _zero16()
    for j in range(RN):
        pltpu.sync_copy(zb, acc.at[pl.ds(r0 + j * RCH, RCH)])
    plsc.subcore_barrier()

    def chunk(i, carry):
        pltpu.sync_copy(ew_a.at[i], acc.at[dst_a.at[i]], add=True)
        return carry

    lax.fori_loop(0, CPT, chunk, 0)
    plsc.subcore_barrier()

    @pl.when(c == 0)
    def _():
        for j in range(RN):
            sl = pl.ds(r0 + j * RCH, RCH)
            pltpu.sync_copy(acc.at[sl], out0.at[sl])

    @pl.when(c == 1)
    def _():
        for j in range(RN):
            sl = pl.ds(r0 + j * RCH, RCH)
            pltpu.sync_copy(acc.at[sl], out1.at[sl])


_deg = pl.kernel(
    _deg_body,
    out_type=[jax.ShapeDtypeStruct((N_PAD,), jnp.float32)] * 2,
    mesh=_MESH,
    scratch_types=[
        pltpu.VMEM((CPT, K), jnp.int32),
        pltpu.VMEM((CPT, K), jnp.float32),
        pltpu.VMEM((K,), jnp.float32),
        pltpu.VMEM_SHARED((N_PAD,), jnp.float32),
    ],
    compiler_params=_SC_PARAMS,
)


def _make_scatter(F):
    nz = F // 16

    def body(hp_hbm, src_hbm, dst_hbm, ew_hbm, out0, out1,
             src_a, dst_a, ew_a, rows_v, acc, sem):
        c = lax.axis_index("c")
        s = lax.axis_index("s")
        wid = s * NC + c
        r0 = s * RPT
        pltpu.sync_copy(src_hbm.at[pl.ds(wid * CPT, CPT)], src_a)
        pltpu.sync_copy(dst_hbm.at[pl.ds(wid * CPT, CPT)], dst_a)
        pltpu.sync_copy(ew_hbm.at[pl.ds(wid * CPT, CPT)], ew_a)

        def zrow(e, carry):
            for j in range(nz):
                rows_v[e, pl.ds(j * 16, 16)] = _zero16()
            return carry

        lax.fori_loop(0, K, zrow, 0)
        for j in range(RN):
            pltpu.sync_copy(rows_v, acc.at[pl.ds(r0 + j * RCH, RCH)])
        plsc.subcore_barrier()

        def chunk(i, carry):
            pltpu.async_copy(hp_hbm.at[src_a.at[i]], rows_v, sem).wait()

            def escale(g, cc):
                w16 = ew_a[i, pl.ds(g * 16, 16)]
                for l in range(16):
                    w = w16[l]
                    e = g * 16 + l
                    for j in range(nz):
                        sl = pl.ds(j * 16, 16)
                        rows_v[e, sl] = rows_v[e, sl] * w
                return cc

            lax.fori_loop(0, K // 16, escale, 0)
            pltpu.sync_copy(rows_v, acc.at[dst_a.at[i]], add=True)
            return carry

        lax.fori_loop(0, CPT, chunk, 0)
        plsc.subcore_barrier()

        @pl.when(c == 0)
        def _():
            for j in range(RN):
                sl = pl.ds(r0 + j * RCH, RCH)
                pltpu.sync_copy(acc.at[sl], out0.at[sl])

        @pl.when(c == 1)
        def _():
            for j in range(RN):
                sl = pl.ds(r0 + j * RCH, RCH)
                pltpu.sync_copy(acc.at[sl], out1.at[sl])

    return pl.kernel(
        body,
        out_type=[jax.ShapeDtypeStruct((N_PAD, F), jnp.float32)] * 2,
        mesh=_MESH,
        scratch_types=[
            pltpu.VMEM((CPT, K), jnp.int32),
            pltpu.VMEM((CPT, K), jnp.int32),
            pltpu.VMEM((CPT, K), jnp.float32),
            pltpu.VMEM((K, F), jnp.float32),
            pltpu.VMEM_SHARED((N_PAD, F), jnp.float32),
            pltpu.SemaphoreType.DMA,
        ],
        compiler_params=_SC_PARAMS,
    )


_scatter128 = _make_scatter(128)
_scatter64 = _make_scatter(64)
_scatter32 = _make_scatter(32)


def _dis_body(d0_ref, d1_ref, o_ref):
    o_ref[...] = lax.rsqrt(d0_ref[...] + d1_ref[...] + 1.0)


def _dis(d0, d1):
    return pl.pallas_call(
        _dis_body,
        out_shape=jax.ShapeDtypeStruct((N_PAD, 1), jnp.float32),
    )(d0.reshape(N_PAD, 1), d1.reshape(N_PAD, 1))


def _k1_body(x_ref, w_ref, dis_ref, o_ref):
    dis = dis_ref[pl.ds(0, N), :]
    o_ref[...] = (
        jnp.dot(x_ref[...], w_ref[...], preferred_element_type=jnp.float32)
        * dis
    )


def _k1(x, W, dis):
    return pl.pallas_call(
        _k1_body,
        out_shape=jax.ShapeDtypeStruct((N, W.shape[1]), jnp.float32),
    )(x, W, dis)


def _fuse_body(a0_ref, a1_ref, hp_ref, dis_ref, b_ref, w_ref, o_ref):
    dis = dis_ref[pl.ds(0, N), :]
    acc = a0_ref[pl.ds(0, N), :] + a1_ref[pl.ds(0, N), :]
    z = dis * (acc + hp_ref[...]) + b_ref[...]
    a = jnp.maximum(z, 0.0)
    o_ref[...] = (
        jnp.dot(a, w_ref[...], preferred_element_type=jnp.float32) * dis
    )


def _fuse(a0, a1, hp, dis, b, W):
    return pl.pallas_call(
        _fuse_body,
        out_shape=jax.ShapeDtypeStruct((N, W.shape[1]), jnp.float32),
    )(a0, a1, hp, dis, b[None, :], W)


def _final_body(a0_ref, a1_ref, hp_ref, dis_ref, b_ref, o_ref):
    dis = dis_ref[pl.ds(0, N), :]
    acc = a0_ref[pl.ds(0, N), :] + a1_ref[pl.ds(0, N), :]
    o_ref[...] = dis * (acc + hp_ref[...]) + b_ref[...]


def _final(a0, a1, hp, dis, b):
    return pl.pallas_call(
        _final_body,
        out_shape=jax.ShapeDtypeStruct((N, b.shape[0]), jnp.float32),
    )(a0, a1, hp, dis, b[None, :])


def kernel(x, edge_index, edge_attr, W1, b1, W2, b2, W3, b3):
    pad = E_PAD - E
    src = jnp.concatenate([edge_index[0].astype(jnp.int32),
                           jnp.zeros((pad,), jnp.int32)]).reshape(NCHUNKS, K)
    dst = jnp.concatenate([edge_index[1].astype(jnp.int32),
                           jnp.zeros((pad,), jnp.int32)]).reshape(NCHUNKS, K)
    ew = jnp.concatenate([edge_attr,
                          jnp.zeros((pad,), jnp.float32)]).reshape(NCHUNKS, K)

    d0, d1 = _deg(dst, ew)
    dis = _dis(d0, d1)

    hp1 = _k1(x, W1, dis)
    a0, a1 = _scatter128(hp1, src, dst, ew)
    hp2 = _fuse(a0, a1, hp1, dis, b1, W2)
    a0, a1 = _scatter64(hp2, src, dst, ew)
    hp3 = _fuse(a0, a1, hp2, dis, b2, W3)
    a0, a1 = _scatter32(hp3, src, dst, ew)
    return _final(a0, a1, hp3, dis, b3)


# speedup vs baseline: 12.2593x; 1.3837x over previous
import functools

import jax
import jax.numpy as jnp
from jax import lax
from jax.experimental import pallas as pl
from jax.experimental.pallas import tpu as pltpu
from jax.experimental.pallas import tpu_sc as plsc

N = 10000
N_PAD = 10240
E = 320000
NC = 2
NS = 16
K = 128
CPT = 160
NCHUNKS = NS * CPT
E_PAD = NCHUNKS * K
DCPT = NCHUNKS // (NC * NS)
RPT = N_PAD // NS
RCH = 128
RN = RPT // RCH

_MESH = plsc.VectorSubcoreMesh(core_axis_name="c", subcore_axis_name="s")
_SC_PARAMS = pltpu.CompilerParams(use_tc_tiling_on_sc=False)


def _zero16():
    return jnp.zeros((16,), jnp.float32)


def _deg_body(dst_hbm, ew_hbm, out0, out1, dst_a, ew_a, zb, acc):
    c = lax.axis_index("c")
    s = lax.axis_index("s")
    wid = s * NC + c
    r0 = s * RPT
    pltpu.sync_copy(dst_hbm.at[pl.ds(wid * DCPT, DCPT)], dst_a)
    pltpu.sync_copy(ew_hbm.at[pl.ds(wid * DCPT, DCPT)], ew_a)
    for j in range(K // 16):
        zb[pl.ds(j * 16, 16)] = _zero16()
    for j in range(RN):
        pltpu.sync_copy(zb, acc.at[pl.ds(r0 + j * RCH, RCH)])
    plsc.subcore_barrier()

    def chunk(i, carry):
        pltpu.sync_copy(ew_a.at[i], acc.at[dst_a.at[i]], add=True)
        return carry

    lax.fori_loop(0, DCPT, chunk, 0)
    plsc.subcore_barrier()

    @pl.when(c == 0)
    def _():
        for j in range(RN):
            sl = pl.ds(r0 + j * RCH, RCH)
            pltpu.sync_copy(acc.at[sl], out0.at[sl])

    @pl.when(c == 1)
    def _():
        for j in range(RN):
            sl = pl.ds(r0 + j * RCH, RCH)
            pltpu.sync_copy(acc.at[sl], out1.at[sl])


_deg = pl.kernel(
    _deg_body,
    out_type=[jax.ShapeDtypeStruct((N_PAD,), jnp.float32)] * 2,
    mesh=_MESH,
    scratch_types=[
        pltpu.VMEM((DCPT, K), jnp.int32),
        pltpu.VMEM((DCPT, K), jnp.float32),
        pltpu.VMEM((K,), jnp.float32),
        pltpu.VMEM_SHARED((N_PAD,), jnp.float32),
    ],
    compiler_params=_SC_PARAMS,
)


def _make_scatter(FFULL):
    FH = FFULL // 2
    nz = FH // 16

    def body(hp0_hbm, hp1_hbm, src_hbm, dst_hbm, ew_hbm, out,
             src_a, dst_a, ew_a, rb0, rb1, acc, sg0, sg1):
        c = lax.axis_index("c")
        s = lax.axis_index("s")
        r0 = s * RPT
        pltpu.sync_copy(src_hbm.at[pl.ds(s * CPT, CPT)], src_a)
        pltpu.sync_copy(dst_hbm.at[pl.ds(s * CPT, CPT)], dst_a)
        pltpu.sync_copy(ew_hbm.at[pl.ds(s * CPT, CPT)], ew_a)

        def zrow(e, carry):
            for j in range(nz):
                rb0[e, pl.ds(j * 16, 16)] = _zero16()
            return carry

        lax.fori_loop(0, K, zrow, 0)
        for j in range(RN):
            pltpu.sync_copy(rb0, acc.at[pl.ds(r0 + j * RCH, RCH)])

        rows = (rb0, rb1)
        sg = (sg0, sg1)

        def gather(i, b):
            @pl.when(c == 0)
            def _():
                pltpu.async_copy(hp0_hbm.at[src_a.at[i]], rows[b], sg[b])

            @pl.when(c == 1)
            def _():
                pltpu.async_copy(hp1_hbm.at[src_a.at[i]], rows[b], sg[b])

        gather(0, 0)
        plsc.subcore_barrier()

        def step(i, b):
            bn = 1 - b
            gather(jnp.minimum(i + 1, CPT - 1), bn)
            pltpu.make_async_copy(
                hp0_hbm.at[src_a.at[i]], rows[b], sg[b]).wait()

            def escale(g, cc):
                w16 = ew_a[i, pl.ds(g * 16, 16)]
                for l in range(16):
                    w = w16[l]
                    e = g * 16 + l
                    for j in range(nz):
                        sl = pl.ds(j * 16, 16)
                        rows[b][e, sl] = rows[b][e, sl] * w
                return cc

            lax.fori_loop(0, K // 16, escale, 0)
            pltpu.sync_copy(rows[b], acc.at[dst_a.at[i]], add=True)

        def chunk(t, carry):
            step(2 * t, 0)
            step(2 * t + 1, 1)
            return carry

        lax.fori_loop(0, CPT // 2, chunk, 0)
        pltpu.make_async_copy(hp0_hbm.at[src_a.at[0]], rb0, sg0).wait()
        plsc.subcore_barrier()

        for j in range(RN):
            sl = pl.ds(r0 + j * RCH, RCH)
            pltpu.sync_copy(acc.at[sl], out.at[sl, pl.ds(c * FH, FH)])

    return pl.kernel(
        body,
        out_type=jax.ShapeDtypeStruct((N_PAD, FFULL), jnp.float32),
        mesh=_MESH,
        scratch_types=[
            pltpu.VMEM((CPT, K), jnp.int32),
            pltpu.VMEM((CPT, K), jnp.int32),
            pltpu.VMEM((CPT, K), jnp.float32),
            pltpu.VMEM((K, FH), jnp.float32),
            pltpu.VMEM((K, FH), jnp.float32),
            pltpu.VMEM_SHARED((N_PAD, FH), jnp.float32),
            pltpu.SemaphoreType.DMA,
            pltpu.SemaphoreType.DMA,
        ],
        compiler_params=_SC_PARAMS,
    )


_scatter128 = _make_scatter(128)
_scatter64 = _make_scatter(64)
_scatter32 = _make_scatter(32)


def _dis_body(d0_ref, d1_ref, o_ref):
    o_ref[...] = lax.rsqrt(d0_ref[...] + d1_ref[...] + 1.0)


def _dis(d0, d1):
    return pl.pallas_call(
        _dis_body,
        out_shape=jax.ShapeDtypeStruct((N_PAD, 1), jnp.float32),
    )(d0.reshape(N_PAD, 1), d1.reshape(N_PAD, 1))


def _k1_body(x_ref, w_ref, dis_ref, o0_ref, o1_ref):
    dis = dis_ref[pl.ds(0, N), :]
    r = (
        jnp.dot(x_ref[...], w_ref[...], preferred_element_type=jnp.float32)
        * dis
    )
    h = r.shape[1] // 2
    o0_ref[...] = r[:, :h]
    o1_ref[...] = r[:, h:]


def _k1(x, W, dis):
    h = W.shape[1] // 2
    return pl.pallas_call(
        _k1_body,
        out_shape=[jax.ShapeDtypeStruct((N, h), jnp.float32)] * 2,
    )(x, W, dis)


def _fuse_body(acc_ref, hpa_ref, hpb_ref, dis_ref, b_ref, w_ref,
               o0_ref, o1_ref):
    dis = dis_ref[pl.ds(0, N), :]
    hp = jnp.concatenate([hpa_ref[...], hpb_ref[...]], axis=1)
    z = dis * (acc_ref[pl.ds(0, N), :] + hp) + b_ref[...]
    a = jnp.maximum(z, 0.0)
    r = jnp.dot(a, w_ref[...], preferred_element_type=jnp.float32) * dis
    h = r.shape[1] // 2
    o0_ref[...] = r[:, :h]
    o1_ref[...] = r[:, h:]


def _fuse(acc, hpa, hpb, dis, b, W):
    h = W.shape[1] // 2
    return pl.pallas_call(
        _fuse_body,
        out_shape=[jax.ShapeDtypeStruct((N, h), jnp.float32)] * 2,
    )(acc, hpa, hpb, dis, b[None, :], W)


def _final_body(acc_ref, hpa_ref, hpb_ref, dis_ref, b_ref, o_ref):
    dis = dis_ref[pl.ds(0, N), :]
    hp = jnp.concatenate([hpa_ref[...], hpb_ref[...]], axis=1)
    o_ref[...] = dis * (acc_ref[pl.ds(0, N), :] + hp) + b_ref[...]


def _final(acc, hpa, hpb, dis, b):
    return pl.pallas_call(
        _final_body,
        out_shape=jax.ShapeDtypeStruct((N, b.shape[0]), jnp.float32),
    )(acc, hpa, hpb, dis, b[None, :])


def kernel(x, edge_index, edge_attr, W1, b1, W2, b2, W3, b3):
    pad = E_PAD - E
    src = jnp.concatenate([edge_index[0].astype(jnp.int32),
                           jnp.zeros((pad,), jnp.int32)]).reshape(NCHUNKS, K)
    dst = jnp.concatenate([edge_index[1].astype(jnp.int32),
                           jnp.zeros((pad,), jnp.int32)]).reshape(NCHUNKS, K)
    ew = jnp.concatenate([edge_attr,
                          jnp.zeros((pad,), jnp.float32)]).reshape(NCHUNKS, K)

    d0, d1 = _deg(dst, ew)
    dis = _dis(d0, d1)

    hp1a, hp1b = _k1(x, W1, dis)
    acc = _scatter128(hp1a, hp1b, src, dst, ew)
    hp2a, hp2b = _fuse(acc, hp1a, hp1b, dis, b1, W2)
    acc = _scatter64(hp2a, hp2b, src, dst, ew)
    hp3a, hp3b = _fuse(acc, hp2a, hp2b, dis, b2, W3)
    acc = _scatter32(hp3a, hp3b, src, dst, ew)
    return _final(acc, hp3a, hp3b, dis, b3)


# speedup vs baseline: 14.6075x; 1.1915x over previous
import functools

import jax
import jax.numpy as jnp
from jax import lax
from jax.experimental import pallas as pl
from jax.experimental.pallas import tpu as pltpu
from jax.experimental.pallas import tpu_sc as plsc

N = 10000
N_PAD = 10240
E = 320000
NC = 2
NS = 16
K = 128
CPT = 160
NCHUNKS = NS * CPT
E_PAD = NCHUNKS * K
DCPT = NCHUNKS // (NC * NS)
RPT = N_PAD // NS
RCH = 128
RN = RPT // RCH

_MESH = plsc.VectorSubcoreMesh(core_axis_name="c", subcore_axis_name="s")
_SC_PARAMS = pltpu.CompilerParams(use_tc_tiling_on_sc=False)


def _zero16():
    return jnp.zeros((16,), jnp.float32)


def _deg_body(dst_hbm, ew_hbm, out0, out1, dst_a, ew_a, zb, acc):
    c = lax.axis_index("c")
    s = lax.axis_index("s")
    wid = s * NC + c
    r0 = s * RPT
    pltpu.sync_copy(dst_hbm.at[pl.ds(wid * DCPT, DCPT)], dst_a)
    pltpu.sync_copy(ew_hbm.at[pl.ds(wid * DCPT, DCPT)], ew_a)
    for j in range(K // 16):
        zb[pl.ds(j * 16, 16)] = _zero16()
    for j in range(RN):
        pltpu.sync_copy(zb, acc.at[pl.ds(r0 + j * RCH, RCH)])
    plsc.subcore_barrier()

    def chunk(i, carry):
        pltpu.sync_copy(ew_a.at[i], acc.at[dst_a.at[i]], add=True)
        return carry

    lax.fori_loop(0, DCPT, chunk, 0)
    plsc.subcore_barrier()

    @pl.when(c == 0)
    def _():
        for j in range(RN):
            sl = pl.ds(r0 + j * RCH, RCH)
            pltpu.sync_copy(acc.at[sl], out0.at[sl])

    @pl.when(c == 1)
    def _():
        for j in range(RN):
            sl = pl.ds(r0 + j * RCH, RCH)
            pltpu.sync_copy(acc.at[sl], out1.at[sl])


_deg = pl.kernel(
    _deg_body,
    out_type=[jax.ShapeDtypeStruct((N_PAD,), jnp.float32)] * 2,
    mesh=_MESH,
    scratch_types=[
        pltpu.VMEM((DCPT, K), jnp.int32),
        pltpu.VMEM((DCPT, K), jnp.float32),
        pltpu.VMEM((K,), jnp.float32),
        pltpu.VMEM_SHARED((N_PAD,), jnp.float32),
    ],
    compiler_params=_SC_PARAMS,
)


def _make_scatter(FFULL):
    FH = FFULL // 2
    nz = FH // 16

    def body(hp0_hbm, hp1_hbm, src_hbm, dst_hbm, ew_hbm, out,
             src_a, dst_a, ew_a, rb0, rb1, acc, sg0, sg1):
        c = lax.axis_index("c")
        s = lax.axis_index("s")
        r0 = s * RPT
        pltpu.sync_copy(src_hbm.at[pl.ds(s * CPT, CPT)], src_a)
        pltpu.sync_copy(dst_hbm.at[pl.ds(s * CPT, CPT)], dst_a)
        pltpu.sync_copy(ew_hbm.at[pl.ds(s * CPT, CPT)], ew_a)

        def zrow(e, carry):
            for j in range(nz):
                rb0[e, pl.ds(j * 16, 16)] = _zero16()
            return carry

        lax.fori_loop(0, K, zrow, 0)
        for j in range(RN):
            pltpu.sync_copy(rb0, acc.at[pl.ds(r0 + j * RCH, RCH)])

        rows = (rb0, rb1)
        sg = (sg0, sg1)

        def gather(i, b):
            @pl.when(c == 0)
            def _():
                pltpu.async_copy(hp0_hbm.at[src_a.at[i]], rows[b], sg[b])

            @pl.when(c == 1)
            def _():
                pltpu.async_copy(hp1_hbm.at[src_a.at[i]], rows[b], sg[b])

        gather(0, 0)
        plsc.subcore_barrier()

        def step(i, b):
            bn = 1 - b
            gather(jnp.minimum(i + 1, CPT - 1), bn)
            pltpu.make_async_copy(
                hp0_hbm.at[src_a.at[i]], rows[b], sg[b]).wait()

            @plsc.parallel_loop(0, K // 16, unroll=2)
            def _(g):
                w16 = ew_a[i, pl.ds(g * 16, 16)]
                for l in range(16):
                    w = w16[l]
                    e = g * 16 + l
                    vals = [rows[b][e, pl.ds(j * 16, 16)] for j in range(nz)]
                    for j in range(nz):
                        rows[b][e, pl.ds(j * 16, 16)] = vals[j] * w
            pltpu.sync_copy(rows[b], acc.at[dst_a.at[i]], add=True)

        def chunk(t, carry):
            step(2 * t, 0)
            step(2 * t + 1, 1)
            return carry

        lax.fori_loop(0, CPT // 2, chunk, 0)
        pltpu.make_async_copy(hp0_hbm.at[src_a.at[0]], rb0, sg0).wait()
        plsc.subcore_barrier()

        for j in range(RN):
            sl = pl.ds(r0 + j * RCH, RCH)
            pltpu.sync_copy(acc.at[sl], out.at[sl, pl.ds(c * FH, FH)])

    return pl.kernel(
        body,
        out_type=jax.ShapeDtypeStruct((N_PAD, FFULL), jnp.float32),
        mesh=_MESH,
        scratch_types=[
            pltpu.VMEM((CPT, K), jnp.int32),
            pltpu.VMEM((CPT, K), jnp.int32),
            pltpu.VMEM((CPT, K), jnp.float32),
            pltpu.VMEM((K, FH), jnp.float32),
            pltpu.VMEM((K, FH), jnp.float32),
            pltpu.VMEM_SHARED((N_PAD, FH), jnp.float32),
            pltpu.SemaphoreType.DMA,
            pltpu.SemaphoreType.DMA,
        ],
        compiler_params=_SC_PARAMS,
    )


_scatter128 = _make_scatter(128)
_scatter64 = _make_scatter(64)
_scatter32 = _make_scatter(32)


def _dis_body(d0_ref, d1_ref, o_ref):
    o_ref[...] = lax.rsqrt(d0_ref[...] + d1_ref[...] + 1.0)


def _dis(d0, d1):
    return pl.pallas_call(
        _dis_body,
        out_shape=jax.ShapeDtypeStruct((N_PAD, 1), jnp.float32),
    )(d0.reshape(N_PAD, 1), d1.reshape(N_PAD, 1))


def _k1_body(x_ref, w_ref, dis_ref, o0_ref, o1_ref):
    dis = dis_ref[pl.ds(0, N), :]
    r = (
        jnp.dot(x_ref[...], w_ref[...], preferred_element_type=jnp.float32)
        * dis
    )
    h = r.shape[1] // 2
    o0_ref[...] = r[:, :h]
    o1_ref[...] = r[:, h:]


def _k1(x, W, dis):
    h = W.shape[1] // 2
    return pl.pallas_call(
        _k1_body,
        out_shape=[jax.ShapeDtypeStruct((N, h), jnp.float32)] * 2,
    )(x, W, dis)


def _fuse_body(acc_ref, hpa_ref, hpb_ref, dis_ref, b_ref, w_ref,
               o0_ref, o1_ref):
    dis = dis_ref[pl.ds(0, N), :]
    hp = jnp.concatenate([hpa_ref[...], hpb_ref[...]], axis=1)
    z = dis * (acc_ref[pl.ds(0, N), :] + hp) + b_ref[...]
    a = jnp.maximum(z, 0.0)
    r = jnp.dot(a, w_ref[...], preferred_element_type=jnp.float32) * dis
    h = r.shape[1] // 2
    o0_ref[...] = r[:, :h]
    o1_ref[...] = r[:, h:]


def _fuse(acc, hpa, hpb, dis, b, W):
    h = W.shape[1] // 2
    return pl.pallas_call(
        _fuse_body,
        out_shape=[jax.ShapeDtypeStruct((N, h), jnp.float32)] * 2,
    )(acc, hpa, hpb, dis, b[None, :], W)


def _final_body(acc_ref, hpa_ref, hpb_ref, dis_ref, b_ref, o_ref):
    dis = dis_ref[pl.ds(0, N), :]
    hp = jnp.concatenate([hpa_ref[...], hpb_ref[...]], axis=1)
    o_ref[...] = dis * (acc_ref[pl.ds(0, N), :] + hp) + b_ref[...]


def _final(acc, hpa, hpb, dis, b):
    return pl.pallas_call(
        _final_body,
        out_shape=jax.ShapeDtypeStruct((N, b.shape[0]), jnp.float32),
    )(acc, hpa, hpb, dis, b[None, :])


def kernel(x, edge_index, edge_attr, W1, b1, W2, b2, W3, b3):
    pad = E_PAD - E
    src = jnp.concatenate([edge_index[0].astype(jnp.int32),
                           jnp.zeros((pad,), jnp.int32)]).reshape(NCHUNKS, K)
    dst = jnp.concatenate([edge_index[1].astype(jnp.int32),
                           jnp.zeros((pad,), jnp.int32)]).reshape(NCHUNKS, K)
    ew = jnp.concatenate([edge_attr,
                          jnp.zeros((pad,), jnp.float32)]).reshape(NCHUNKS, K)

    d0, d1 = _deg(dst, ew)
    dis = _dis(d0, d1)

    hp1a, hp1b = _k1(x, W1, dis)
    acc = _scatter128(hp1a, hp1b, src, dst, ew)
    hp2a, hp2b = _fuse(acc, hp1a, hp1b, dis, b1, W2)
    acc = _scatter64(hp2a, hp2b, src, dst, ew)
    hp3a, hp3b = _fuse(acc, hp2a, hp2b, dis, b2, W3)
    acc = _scatter32(hp3a, hp3b, src, dst, ew)
    return _final(acc, hp3a, hp3b, dis, b3)


# speedup vs baseline: 15.3250x; 1.0491x over previous
import functools

import jax
import jax.numpy as jnp
from jax import lax
from jax.experimental import pallas as pl
from jax.experimental.pallas import tpu as pltpu
from jax.experimental.pallas import tpu_sc as plsc

N = 10000
N_PAD = 10240
E = 320000
NC = 2
NS = 16
K = 128
CPT = 160
NCHUNKS = NS * CPT
E_PAD = NCHUNKS * K
DCPT = NCHUNKS // (NC * NS)
RPT = N_PAD // NS
RCH = 128
RN = RPT // RCH

_MESH = plsc.VectorSubcoreMesh(core_axis_name="c", subcore_axis_name="s")
_SC_PARAMS = pltpu.CompilerParams(use_tc_tiling_on_sc=False)


def _zero16():
    return jnp.zeros((16,), jnp.float32)


def _deg_body(dst_hbm, ew_hbm, out0, out1, dst_a, ew_a, zb, acc):
    c = lax.axis_index("c")
    s = lax.axis_index("s")
    wid = s * NC + c
    r0 = s * RPT
    pltpu.sync_copy(dst_hbm.at[pl.ds(wid * DCPT, DCPT)], dst_a)
    pltpu.sync_copy(ew_hbm.at[pl.ds(wid * DCPT, DCPT)], ew_a)
    for j in range(K // 16):
        zb[pl.ds(j * 16, 16)] = _zero16()
    for j in range(RN):
        pltpu.sync_copy(zb, acc.at[pl.ds(r0 + j * RCH, RCH)])
    plsc.subcore_barrier()

    def chunk(i, carry):
        pltpu.sync_copy(ew_a.at[i], acc.at[dst_a.at[i]], add=True)
        return carry

    lax.fori_loop(0, DCPT, chunk, 0)
    plsc.subcore_barrier()

    @pl.when(c == 0)
    def _():
        for j in range(RN):
            sl = pl.ds(r0 + j * RCH, RCH)
            pltpu.sync_copy(acc.at[sl], out0.at[sl])

    @pl.when(c == 1)
    def _():
        for j in range(RN):
            sl = pl.ds(r0 + j * RCH, RCH)
            pltpu.sync_copy(acc.at[sl], out1.at[sl])


_deg = pl.kernel(
    _deg_body,
    out_type=[jax.ShapeDtypeStruct((N_PAD,), jnp.float32)] * 2,
    mesh=_MESH,
    scratch_types=[
        pltpu.VMEM((DCPT, K), jnp.int32),
        pltpu.VMEM((DCPT, K), jnp.float32),
        pltpu.VMEM((K,), jnp.float32),
        pltpu.VMEM_SHARED((N_PAD,), jnp.float32),
    ],
    compiler_params=_SC_PARAMS,
)


def _make_scatter(FFULL):
    FH = FFULL // 2
    nz = FH // 16

    def body(hp0_hbm, hp1_hbm, src_hbm, dst_hbm, ew_hbm, out,
             src_a, dst_a, ew_a, rb0, rb1, rb2, acc,
             sg0, sg1, sg2, ss0, ss1, ss2):
        c = lax.axis_index("c")
        s = lax.axis_index("s")
        r0 = s * RPT
        pltpu.sync_copy(src_hbm.at[pl.ds(s * CPT, CPT)], src_a)
        pltpu.sync_copy(dst_hbm.at[pl.ds(s * CPT, CPT)], dst_a)
        pltpu.sync_copy(ew_hbm.at[pl.ds(s * CPT, CPT)], ew_a)

        def zrow(e, carry):
            for j in range(nz):
                rb0[e, pl.ds(j * 16, 16)] = _zero16()
            return carry

        lax.fori_loop(0, K, zrow, 0)
        for j in range(RN):
            pltpu.sync_copy(rb0, acc.at[pl.ds(r0 + j * RCH, RCH)])

        rows = (rb0, rb1, rb2)
        sg = (sg0, sg1, sg2)
        ss = (ss0, ss1, ss2)

        def gather(i, b):
            @pl.when(c == 0)
            def _():
                pltpu.async_copy(hp0_hbm.at[src_a.at[i]], rows[b], sg[b])

            @pl.when(c == 1)
            def _():
                pltpu.async_copy(hp1_hbm.at[src_a.at[i]], rows[b], sg[b])

        gather(0, 0)
        plsc.subcore_barrier()

        def step(i, b, wait_scat, issue_next):
            bn = (b + 1) % 3
            if wait_scat:
                pltpu.make_async_copy(
                    rows[bn], acc.at[dst_a.at[0]], ss[bn]).wait()
            if issue_next:
                gather(i + 1, bn)
            pltpu.make_async_copy(
                hp0_hbm.at[src_a.at[i]], rows[b], sg[b]).wait()

            @plsc.parallel_loop(0, K // 16, unroll=2)
            def _(g):
                w16 = ew_a[i, pl.ds(g * 16, 16)]
                for l in range(16):
                    w = w16[l]
                    e = g * 16 + l
                    vals = [rows[b][e, pl.ds(j * 16, 16)] for j in range(nz)]
                    for j in range(nz):
                        rows[b][e, pl.ds(j * 16, 16)] = vals[j] * w

            pltpu.async_copy(rows[b], acc.at[dst_a.at[i]], ss[b], add=True)

        step(0, 0, False, True)
        step(1, 1, False, True)

        def chunk(t, carry):
            for u in range(3):
                step(2 + 3 * t + u, (2 + u) % 3, True, True)
            return carry

        lax.fori_loop(0, (CPT - 4) // 3, chunk, 0)
        step(CPT - 2, (CPT - 2) % 3, True, True)
        step(CPT - 1, (CPT - 1) % 3, True, False)
        pltpu.make_async_copy(
            rows[(CPT - 2) % 3], acc.at[dst_a.at[0]],
            ss[(CPT - 2) % 3]).wait()
        pltpu.make_async_copy(
            rows[(CPT - 1) % 3], acc.at[dst_a.at[0]],
            ss[(CPT - 1) % 3]).wait()
        plsc.subcore_barrier()

        for j in range(RN):
            sl = pl.ds(r0 + j * RCH, RCH)
            pltpu.sync_copy(acc.at[sl], out.at[sl, pl.ds(c * FH, FH)])

    return pl.kernel(
        body,
        out_type=jax.ShapeDtypeStruct((N_PAD, FFULL), jnp.float32),
        mesh=_MESH,
        scratch_types=[
            pltpu.VMEM((CPT, K), jnp.int32),
            pltpu.VMEM((CPT, K), jnp.int32),
            pltpu.VMEM((CPT, K), jnp.float32),
            pltpu.VMEM((K, FH), jnp.float32),
            pltpu.VMEM((K, FH), jnp.float32),
            pltpu.VMEM((K, FH), jnp.float32),
            pltpu.VMEM_SHARED((N_PAD, FH), jnp.float32),
            pltpu.SemaphoreType.DMA,
            pltpu.SemaphoreType.DMA,
            pltpu.SemaphoreType.DMA,
            pltpu.SemaphoreType.DMA,
            pltpu.SemaphoreType.DMA,
            pltpu.SemaphoreType.DMA,
        ],
        compiler_params=_SC_PARAMS,
    )


_scatter128 = _make_scatter(128)
_scatter64 = _make_scatter(64)
_scatter32 = _make_scatter(32)


def _dis_body(d0_ref, d1_ref, o_ref):
    o_ref[...] = lax.rsqrt(d0_ref[...] + d1_ref[...] + 1.0)


def _dis(d0, d1):
    return pl.pallas_call(
        _dis_body,
        out_shape=jax.ShapeDtypeStruct((N_PAD, 1), jnp.float32),
    )(d0.reshape(N_PAD, 1), d1.reshape(N_PAD, 1))


def _k1_body(x_ref, w_ref, dis_ref, o0_ref, o1_ref):
    dis = dis_ref[pl.ds(0, N), :]
    r = (
        jnp.dot(x_ref[...], w_ref[...], preferred_element_type=jnp.float32)
        * dis
    )
    h = r.shape[1] // 2
    o0_ref[...] = r[:, :h]
    o1_ref[...] = r[:, h:]


def _k1(x, W, dis):
    h = W.shape[1] // 2
    return pl.pallas_call(
        _k1_body,
        out_shape=[jax.ShapeDtypeStruct((N, h), jnp.float32)] * 2,
    )(x, W, dis)


def _fuse_body(acc_ref, hpa_ref, hpb_ref, dis_ref, b_ref, w_ref,
               o0_ref, o1_ref):
    dis = dis_ref[pl.ds(0, N), :]
    hp = jnp.concatenate([hpa_ref[...], hpb_ref[...]], axis=1)
    z = dis * (acc_ref[pl.ds(0, N), :] + hp) + b_ref[...]
    a = jnp.maximum(z, 0.0)
    r = jnp.dot(a, w_ref[...], preferred_element_type=jnp.float32) * dis
    h = r.shape[1] // 2
    o0_ref[...] = r[:, :h]
    o1_ref[...] = r[:, h:]


def _fuse(acc, hpa, hpb, dis, b, W):
    h = W.shape[1] // 2
    return pl.pallas_call(
        _fuse_body,
        out_shape=[jax.ShapeDtypeStruct((N, h), jnp.float32)] * 2,
    )(acc, hpa, hpb, dis, b[None, :], W)


def _final_body(acc_ref, hpa_ref, hpb_ref, dis_ref, b_ref, o_ref):
    dis = dis_ref[pl.ds(0, N), :]
    hp = jnp.concatenate([hpa_ref[...], hpb_ref[...]], axis=1)
    o_ref[...] = dis * (acc_ref[pl.ds(0, N), :] + hp) + b_ref[...]


def _final(acc, hpa, hpb, dis, b):
    return pl.pallas_call(
        _final_body,
        out_shape=jax.ShapeDtypeStruct((N, b.shape[0]), jnp.float32),
    )(acc, hpa, hpb, dis, b[None, :])


def kernel(x, edge_index, edge_attr, W1, b1, W2, b2, W3, b3):
    pad = E_PAD - E
    src = jnp.concatenate([edge_index[0].astype(jnp.int32),
                           jnp.zeros((pad,), jnp.int32)]).reshape(NCHUNKS, K)
    dst = jnp.concatenate([edge_index[1].astype(jnp.int32),
                           jnp.zeros((pad,), jnp.int32)]).reshape(NCHUNKS, K)
    ew = jnp.concatenate([edge_attr,
                          jnp.zeros((pad,), jnp.float32)]).reshape(NCHUNKS, K)

    d0, d1 = _deg(dst, ew)
    dis = _dis(d0, d1)

    hp1a, hp1b = _k1(x, W1, dis)
    acc = _scatter128(hp1a, hp1b, src, dst, ew)
    hp2a, hp2b = _fuse(acc, hp1a, hp1b, dis, b1, W2)
    acc = _scatter64(hp2a, hp2b, src, dst, ew)
    hp3a, hp3b = _fuse(acc, hp2a, hp2b, dis, b2, W3)
    acc = _scatter32(hp3a, hp3b, src, dst, ew)
    return _final(acc, hp3a, hp3b, dis, b3)
